# baseline (speedup 1.0000x reference)
import functools

import jax
import jax.numpy as jnp
from jax import lax
from jax.experimental import pallas as pl
from jax.experimental.pallas import tpu as pltpu
from jax.experimental.pallas import tpu_sc as plsc

_EPS = 1e-6
_NW = 32
_CT = 6
_C = 768
_L = 16
_NO = 6

_GDN = lax.GatherDimensionNumbers(
    offset_dims=(), collapsed_slice_dims=(0,), start_index_map=(0,)
)


def _shuf(v, perm2d):
    return lax.gather(
        v,
        perm2d,
        dimension_numbers=_GDN,
        slice_sizes=(1,),
        mode=lax.GatherScatterMode.PROMISE_IN_BOUNDS,
    )


def _rsqrt_newton(v):
    i = lax.bitcast_convert_type(v, jnp.int32)
    i = jnp.int32(0x5F3759DF) - lax.shift_right_logical(i, 1)
    y = lax.bitcast_convert_type(i, jnp.float32)
    for _ in range(3):
        y = y * (1.5 - 0.5 * v * y * y)
    return y


def _allsum(v, perms):
    for perm in perms:
        v = v + _shuf(v, perm)
    return v


def _make_sc_call(OCT):
    per_w = OCT // _NW
    nchunk = per_w // _NO
    mesh = plsc.VectorSubcoreMesh(core_axis_name="c", subcore_axis_name="s")

    @functools.partial(
        pl.kernel,
        mesh=mesh,
        out_type=jax.ShapeDtypeStruct((OCT, _CT, 8, 128), jnp.float32),
        scratch_types=[
            pltpu.VMEM((_NO, _CT, 8, 128), jnp.float32),
            pltpu.VMEM((_NO, _CT, 8, 128), jnp.float32),
            pltpu.VMEM((_NO, _CT, 8, 128), jnp.float32),
            pltpu.VMEM((24, 128), jnp.float32),
            pltpu.SemaphoreType.DMA,
            pltpu.SemaphoreType.DMA,
            pltpu.SemaphoreType.DMA,
            pltpu.SemaphoreType.DMA,
            pltpu.SemaphoreType.DMA,
            pltpu.SemaphoreType.DMA,
        ],
    )
    def call(x_hbm, m_hbm, o_hbm, xbuf0, xbuf1, xbuf2, mbuf,
             si0, si1, si2, so0, so1, so2):
        wid = lax.axis_index("s") * 2 + lax.axis_index("c")
        base = wid * per_w
        iota = lax.iota(jnp.int32, _L)
        perms = [jnp.reshape(iota ^ k, (_L, 1)) for k in (8, 4, 2, 1)]
        splats = [jnp.reshape((iota & 0) + p, (_L, 1)) for p in range(8)]

        pltpu.sync_copy(m_hbm.at[wid], mbuf)

        xb = (xbuf0, xbuf1, xbuf2)
        si = (si0, si1, si2)
        so = (so0, so1, so2)

        def start_in(ci, p):
            pltpu.async_copy(x_hbm.at[pl.ds(base + ci * _NO, _NO)], xb[p], si[p])

        def wait_in(p):
            pltpu.make_async_copy(
                x_hbm.at[pl.ds(base, _NO)], xb[p], si[p]).wait()

        def start_out(ci, p):
            pltpu.async_copy(xb[p], o_hbm.at[pl.ds(base + ci * _NO, _NO)], so[p])

        def wait_out(p):
            pltpu.make_async_copy(
                xb[p], o_hbm.at[pl.ds(base, _NO)], so[p]).wait()

        def compute(ci, p):
            xr = orr = xb[p]

            def octet_body(o, carry):
                oc = ci * _NO + o
                mv = mbuf[oc // 8, pl.ds((oc % 8) * 8, 16)]
                for p8 in range(8):
                    a1 = [None] * 6
                    a2 = [None] * 6
                    for j in range(48):
                        v = xr[o, j // 8, p8, pl.ds((j % 8) * _L, _L)]
                        sq = v * v
                        k = j % 6
                        if j < 6:
                            a1[k] = v
                            a2[k] = sq
                        else:
                            a1[k] = a1[k] + v
                            a2[k] = a2[k] + sq
                    s1v = (a1[0] + a1[1]) + (a1[2] + a1[3]) + (a1[4] + a1[5])
                    s2v = (a2[0] + a2[1]) + (a2[2] + a2[3]) + (a2[4] + a2[5])
                    mean = _allsum(s1v, perms) * (1.0 / _C)
                    var = _allsum(s2v, perms) * (1.0 / _C) - mean * mean
                    m = _shuf(mv, splats[p8])
                    s = _rsqrt_newton(var + _EPS) * m
                    for j in range(48):
                        v = xr[o, j // 8, p8, pl.ds((j % 8) * _L, _L)]
                        orr[o, j // 8, p8, pl.ds((j % 8) * _L, _L)] = (v - mean) * s
                return carry

            lax.fori_loop(0, _NO, octet_body, 0)

        start_in(0, 0)
        start_in(1, 1)
        wait_in(0); compute(0, 0); start_out(0, 0); start_in(2, 2)

        def tri_body(j3, carry):
            for t in range(3):
                j = 1 + 3 * j3 + t
                b = (1 + t) % 3
                wait_in(b)
                compute(j, b)
                start_out(j, b)
                wait_out(t % 3)
                start_in(j + 2, t % 3)
            return carry

        lax.fori_loop(0, (nchunk - 3) // 3, tri_body, 0)
        b = (nchunk - 2) % 3
        wait_in(b); compute(nchunk - 2, b); start_out(nchunk - 2, b)
        b = (nchunk - 1) % 3
        wait_in(b); compute(nchunk - 1, b); start_out(nchunk - 1, b)
        wait_out(0)
        wait_out(1)
        wait_out(2)

    return call


def kernel(x, active, ln_weight, ln_bias):
    B, C, H, W = x.shape
    sh = H // active.shape[2]
    sw = W // active.shape[3]
    a = active[:, 0].astype(jnp.float32)
    mask = jnp.repeat(jnp.repeat(a, sh, axis=1), sw, axis=2)
    OCT = B * H * (W // 8)
    maskv = (mask != 0.0).astype(jnp.float32).reshape(_NW, OCT // _NW // 8, 64)
    maskv = jnp.pad(maskv, ((0, 0), (0, 24 - OCT // _NW // 8), (0, 64)))

    xv = jnp.transpose(x, (0, 2, 3, 1))
    xv = xv.reshape(B, H, W // 8, 8, C // 128, 128)
    xv = jnp.transpose(xv, (0, 1, 2, 4, 3, 5))
    xv = xv.reshape(OCT, _CT, 8, 128)

    out = _make_sc_call(OCT)(xv, maskv)

    out = out.reshape(B, H, W // 8, C // 128, 8, 128)
    out = jnp.transpose(out, (0, 1, 2, 4, 3, 5))
    out = out.reshape(B, H, W, C)
    return jnp.transpose(out, (0, 3, 1, 2))
